# Initial kernel scaffold; baseline (speedup 1.0000x reference)
import jax
import jax.numpy as jnp
from jax.experimental import pallas as pl


def kernel(x, edge_index, input_idx, lin_W, lin_b, bn_gamma, bn_beta, ggc_weight, gru_Wih, gru_Whh, gru_bih, gru_bhh, mlp_W1, mlp_b1, mlp_W2, mlp_b2, mlp_W3, mlp_b3):
    raise NotImplementedError("write your pallas kernel here")


# speedup vs baseline: 4.3929x; 4.3929x over previous
import functools

import jax
import jax.numpy as jnp
from jax import lax
from jax.experimental import pallas as pl
from jax.experimental.pallas import tpu as pltpu
from jax.experimental.pallas import tpu_sc as plsc

N_NODES = 10000
INT_D = 256
N_EDGES = 320000
LAYER_N = 4
MIX_N = 4
MAX_N = 100

ROW_BLK = 1000
GRID = N_NODES // ROW_BLK

SC_CORES = 2
SC_TILES = 16
CHUNK = 128
CHUNKS_PER_TILE = -(-N_EDGES // (CHUNK * SC_TILES))
N_CHUNKS = CHUNKS_PER_TILE * SC_TILES
E_PAD = N_CHUNKS * CHUNK
ACC_ROWS = N_NODES + SC_TILES
ROWS_PER_TILE = 624
ROWS_LAST = N_NODES - ROWS_PER_TILE * (SC_TILES - 1)
QCOLS = 64
NQ = INT_D // QCOLS


def _lin_stats_body(x_ref, wt_ref, b_ref, h_ref, stats_ref):
    h = jnp.dot(x_ref[...], wt_ref[...], preferred_element_type=jnp.float32)
    h = h + b_ref[...]
    h_ref[...] = h
    s = jnp.sum(h, axis=0, keepdims=True)
    ss = jnp.sum(h * h, axis=0, keepdims=True)
    blk = jnp.concatenate([s, ss, jnp.zeros((6, INT_D), jnp.float32)], axis=0)

    @pl.when(pl.program_id(0) == 0)
    def _():
        stats_ref[...] = blk

    @pl.when(pl.program_id(0) != 0)
    def _():
        stats_ref[...] = stats_ref[...] + blk


def _lin_stats(x, lin_Wt, lin_b2):
    return pl.pallas_call(
        _lin_stats_body,
        grid=(GRID,),
        in_specs=[
            pl.BlockSpec((ROW_BLK, 128), lambda r: (r, 0)),
            pl.BlockSpec((128, INT_D), lambda r: (0, 0)),
            pl.BlockSpec((1, INT_D), lambda r: (0, 0)),
        ],
        out_specs=[
            pl.BlockSpec((ROW_BLK, INT_D), lambda r: (r, 0)),
            pl.BlockSpec((8, INT_D), lambda r: (0, 0)),
        ],
        out_shape=[
            jax.ShapeDtypeStruct((N_NODES, INT_D), jnp.float32),
            jax.ShapeDtypeStruct((8, INT_D), jnp.float32),
        ],
    )(x, lin_Wt, lin_b2)


def _bn_m0_body(h_ref, stats_ref, g_ref, b_ref, w0_ref, hn_ref, m2_ref):
    mean = stats_ref[0:1, :] * (1.0 / N_NODES)
    ex2 = stats_ref[1:2, :] * (1.0 / N_NODES)
    var = ex2 - mean * mean
    rstd = lax.rsqrt(var + 1e-5)
    hn = (h_ref[...] - mean) * (rstd * g_ref[...]) + b_ref[...]
    hn_ref[...] = hn
    m = jnp.dot(hn, w0_ref[...], preferred_element_type=jnp.float32)
    m2_ref[...] = jnp.stack(
        [m[:, q * QCOLS:(q + 1) * QCOLS] for q in range(NQ)], axis=0)


def _bn_m0(h, stats, gamma2, beta2, w0):
    return pl.pallas_call(
        _bn_m0_body,
        grid=(GRID,),
        in_specs=[
            pl.BlockSpec((ROW_BLK, INT_D), lambda r: (r, 0)),
            pl.BlockSpec((8, INT_D), lambda r: (0, 0)),
            pl.BlockSpec((1, INT_D), lambda r: (0, 0)),
            pl.BlockSpec((1, INT_D), lambda r: (0, 0)),
            pl.BlockSpec((INT_D, INT_D), lambda r: (0, 0)),
        ],
        out_specs=[
            pl.BlockSpec((ROW_BLK, INT_D), lambda r: (r, 0)),
            pl.BlockSpec((NQ, ROW_BLK, QCOLS), lambda r: (0, r, 0)),
        ],
        out_shape=[
            jax.ShapeDtypeStruct((N_NODES, INT_D), jnp.float32),
            jax.ShapeDtypeStruct((NQ, N_NODES, QCOLS), jnp.float32),
        ],
    )(h, stats, gamma2, beta2, w0)


NBUF = 4
NQUADS = CHUNKS_PER_TILE // NBUF
TAIL = CHUNKS_PER_TILE - NQUADS * NBUF


def _sc_scatter_body(m_ref, src_ref, dst_ref, zeros_ref, out_ref,
                     srcv0, srcv1, srcv2, srcv3, dstv0, dstv1, dstv2, dstv3,
                     rows0, rows1, rows2, rows3, bounce, accum,
                     ssem, gsem, dsem):
    c = lax.axis_index("c")
    s = lax.axis_index("s")
    srcvs = (srcv0, srcv1, srcv2, srcv3)
    dstvs = (dstv0, dstv1, dstv2, dstv3)
    rowss = (rows0, rows1, rows2, rows3)

    def zero_stripe():
        @pl.when(s < SC_TILES - 1)
        def _():
            pltpu.sync_copy(zeros_ref.at[pl.ds(0, ROWS_PER_TILE)],
                            accum.at[pl.ds(s * ROWS_PER_TILE, ROWS_PER_TILE)])

        @pl.when(s == SC_TILES - 1)
        def _():
            pltpu.sync_copy(
                zeros_ref,
                accum.at[pl.ds((SC_TILES - 1) * ROWS_PER_TILE, ROWS_LAST)])

    def do_group(q_base, base_chunk, nb):
        idescs = []
        for b in range(nb):
            chunk = s * CHUNKS_PER_TILE + base_chunk + b
            sd = pltpu.async_copy(
                src_ref.at[pl.ds(((q_base + c) * N_CHUNKS + chunk) * CHUNK,
                                 CHUNK)],
                srcvs[b], ssem.at[b])
            dd = pltpu.async_copy(dst_ref.at[pl.ds(chunk * CHUNK, CHUNK)],
                                  dstvs[b], dsem.at[b])
            idescs.append((sd, dd))
        gdescs = []
        for b in range(nb):
            idescs[b][0].wait()
            gdescs.append(
                pltpu.async_copy(m_ref.at[srcvs[b]], rowss[b], gsem.at[b]))
        for b in range(nb):
            idescs[b][1].wait()
            gdescs[b].wait()
            pltpu.sync_copy(rowss[b], accum.at[dstvs[b]], add=True)

    def writeout(q):
        @pl.when(s < SC_TILES - 1)
        def _():
            pltpu.sync_copy(accum.at[pl.ds(s * ROWS_PER_TILE, ROWS_PER_TILE)],
                            bounce.at[pl.ds(0, ROWS_PER_TILE)])
            pltpu.sync_copy(
                bounce.at[pl.ds(0, ROWS_PER_TILE)],
                out_ref.at[q, pl.ds(s * ROWS_PER_TILE, ROWS_PER_TILE)])

        @pl.when(s == SC_TILES - 1)
        def _():
            pltpu.sync_copy(
                accum.at[pl.ds((SC_TILES - 1) * ROWS_PER_TILE, ROWS_LAST)],
                bounce)
            pltpu.sync_copy(
                bounce,
                out_ref.at[q, pl.ds((SC_TILES - 1) * ROWS_PER_TILE,
                                    ROWS_LAST)])

    zero_stripe()
    plsc.subcore_barrier()
    for phase in range(2):
        q_base = 2 * phase

        def quad_body(jj, carry, q_base=q_base):
            do_group(q_base, jj * NBUF, NBUF)
            return carry

        lax.fori_loop(0, NQUADS, quad_body, 0)
        do_group(q_base, NQUADS * NBUF, TAIL)
        plsc.subcore_barrier()
        writeout(q_base + c)
        if phase == 0:
            zero_stripe()
            plsc.subcore_barrier()


@functools.lru_cache(maxsize=None)
def _sc_scatter_fn():
    return pl.kernel(
        _sc_scatter_body,
        out_type=jax.ShapeDtypeStruct((NQ, N_NODES, QCOLS), jnp.float32),
        mesh=plsc.VectorSubcoreMesh(core_axis_name="c", subcore_axis_name="s",
                                    num_cores=SC_CORES, num_subcores=SC_TILES),
        scratch_types=(
            [pltpu.VMEM((CHUNK,), jnp.int32)] * 8
            + [pltpu.VMEM((CHUNK, QCOLS), jnp.float32)] * 4
            + [
                pltpu.VMEM((ROWS_LAST, QCOLS), jnp.float32),
                pltpu.VMEM_SHARED((ACC_ROWS, QCOLS), jnp.float32),
                pltpu.SemaphoreType.DMA((NBUF,)),
                pltpu.SemaphoreType.DMA((NBUF,)),
                pltpu.SemaphoreType.DMA((NBUF,)),
            ]
        ),
        compiler_params=pltpu.CompilerParams(use_tc_tiling_on_sc=False),
    )


def _sc_scatter(m_flat, src4, dst_flat, zeros):
    return _sc_scatter_fn()(m_flat, src4, dst_flat, zeros)


def _gru_body(agg_ref, h_ref, wih_ref, whh_ref, bih_ref, bhh_ref,
              *rest, has_next):
    if has_next:
        wn_ref, hn_ref, m2_ref = rest
    else:
        (hn_ref,) = rest
    agg = jnp.concatenate([agg_ref[q] for q in range(NQ)], axis=1)
    h = h_ref[...]
    gi = jnp.dot(agg, wih_ref[...], preferred_element_type=jnp.float32) + bih_ref[...]
    gh = jnp.dot(h, whh_ref[...], preferred_element_type=jnp.float32) + bhh_ref[...]
    r = jax.nn.sigmoid(gi[:, :INT_D] + gh[:, :INT_D])
    z = jax.nn.sigmoid(gi[:, INT_D:2 * INT_D] + gh[:, INT_D:2 * INT_D])
    n = jnp.tanh(gi[:, 2 * INT_D:] + r * gh[:, 2 * INT_D:])
    hn = (1.0 - z) * n + z * h
    hn_ref[...] = hn
    if has_next:
        m = jnp.dot(hn, wn_ref[...], preferred_element_type=jnp.float32)
        m2_ref[...] = jnp.stack(
            [m[:, q * QCOLS:(q + 1) * QCOLS] for q in range(NQ)], axis=0)


def _gru_step(agg4, h, wih_t, whh_t, bih2, bhh2, w_next):
    has_next = w_next is not None
    in_specs = [
        pl.BlockSpec((NQ, ROW_BLK, QCOLS), lambda r: (0, r, 0)),
        pl.BlockSpec((ROW_BLK, INT_D), lambda r: (r, 0)),
        pl.BlockSpec((INT_D, 3 * INT_D), lambda r: (0, 0)),
        pl.BlockSpec((INT_D, 3 * INT_D), lambda r: (0, 0)),
        pl.BlockSpec((1, 3 * INT_D), lambda r: (0, 0)),
        pl.BlockSpec((1, 3 * INT_D), lambda r: (0, 0)),
    ]
    args = [agg4, h, wih_t, whh_t, bih2, bhh2]
    out_specs = [pl.BlockSpec((ROW_BLK, INT_D), lambda r: (r, 0))]
    out_shape = [jax.ShapeDtypeStruct((N_NODES, INT_D), jnp.float32)]
    if has_next:
        in_specs.append(pl.BlockSpec((INT_D, INT_D), lambda r: (0, 0)))
        args.append(w_next)
        out_specs.append(pl.BlockSpec((NQ, ROW_BLK, QCOLS), lambda r: (0, r, 0)))
        out_shape.append(jax.ShapeDtypeStruct((NQ, N_NODES, QCOLS), jnp.float32))
    res = pl.pallas_call(
        functools.partial(_gru_body, has_next=has_next),
        grid=(GRID,),
        in_specs=in_specs,
        out_specs=out_specs,
        out_shape=out_shape,
    )(*args)
    return res if has_next else res[0]


def _mlp_body(h_ref, sel_ref, w1_ref, b1_ref, w2_ref, b2_ref, w3_ref, b3_ref,
              mu_ref, std_ref):
    v = jax.nn.relu(h_ref[...])
    ys = []
    for i in range(MIX_N):
        y1 = jnp.dot(v, w1_ref[i], preferred_element_type=jnp.float32)
        y1 = jax.nn.relu(y1 + b1_ref[i:i + 1, :])
        y2 = jnp.dot(y1, w2_ref[i], preferred_element_type=jnp.float32)
        y2 = jax.nn.relu(y2 + b2_ref[i:i + 1, :])
        y3 = jnp.sum(y2 * w3_ref[i:i + 1, :], axis=1, keepdims=True)
        ys.append(y3 + b3_ref[:, i:i + 1])
    y = jnp.concatenate(ys, axis=1)
    mu_ref[...] = jnp.sum(y * sel_ref[...], axis=1, keepdims=True)
    mean4 = jnp.mean(y, axis=1, keepdims=True)
    var = jnp.sum((y - mean4) ** 2, axis=1, keepdims=True) * (1.0 / (MIX_N - 1))
    std_ref[...] = jnp.sqrt(var + 1e-5)


def _mlp_mix(h, selmask, w1t, b1, w2t, b2, w3v, b3row):
    return pl.pallas_call(
        _mlp_body,
        grid=(GRID,),
        in_specs=[
            pl.BlockSpec((ROW_BLK, INT_D), lambda r: (r, 0)),
            pl.BlockSpec((ROW_BLK, MIX_N), lambda r: (r, 0)),
            pl.BlockSpec((MIX_N, INT_D, INT_D), lambda r: (0, 0, 0)),
            pl.BlockSpec((MIX_N, INT_D), lambda r: (0, 0)),
            pl.BlockSpec((MIX_N, INT_D, INT_D), lambda r: (0, 0, 0)),
            pl.BlockSpec((MIX_N, INT_D), lambda r: (0, 0)),
            pl.BlockSpec((MIX_N, INT_D), lambda r: (0, 0)),
            pl.BlockSpec((1, MIX_N), lambda r: (0, 0)),
        ],
        out_specs=[
            pl.BlockSpec((ROW_BLK, 1), lambda r: (r, 0)),
            pl.BlockSpec((ROW_BLK, 1), lambda r: (r, 0)),
        ],
        out_shape=[
            jax.ShapeDtypeStruct((N_NODES, 1), jnp.float32),
            jax.ShapeDtypeStruct((N_NODES, 1), jnp.float32),
        ],
    )(h, selmask, w1t, b1, w2t, b2, w3v, b3row)


def kernel(x, edge_index, input_idx, lin_W, lin_b, bn_gamma, bn_beta,
           ggc_weight, gru_Wih, gru_Whh, gru_bih, gru_bhh,
           mlp_W1, mlp_b1, mlp_W2, mlp_b2, mlp_W3, mlp_b3):
    lin_Wt = lin_W.T
    lin_b2 = lin_b[None, :]
    gamma2 = bn_gamma[None, :]
    beta2 = bn_beta[None, :]
    wih_t = gru_Wih.T
    whh_t = gru_Whh.T
    bih2 = gru_bih[None, :]
    bhh2 = gru_bhh[None, :]
    w1t = jnp.swapaxes(mlp_W1, 1, 2)
    w2t = jnp.swapaxes(mlp_W2, 1, 2)
    w3v = mlp_W3[:, 0, :]
    b3row = mlp_b3.reshape(1, MIX_N)

    n_pad = E_PAD - N_EDGES
    src_flat = jnp.concatenate(
        [edge_index[0], jnp.zeros((n_pad,), jnp.int32)])
    dst_flat = jnp.concatenate(
        [edge_index[1],
         N_NODES + (jnp.arange(n_pad, dtype=jnp.int32) % SC_TILES)])
    src4 = jnp.concatenate([src_flat + q * N_NODES for q in range(NQ)])
    zeros = jnp.zeros((ROWS_LAST, QCOLS), jnp.float32)

    sel = input_idx % MIX_N
    onehot = (sel[:, None] == jnp.arange(MIX_N)[None, :]).astype(jnp.float32)
    selmask = jnp.repeat(onehot, MAX_N, axis=0)

    H, stats = _lin_stats(x, lin_Wt, lin_b2)
    h, m2 = _bn_m0(H, stats, gamma2, beta2, ggc_weight[0])
    for i in range(LAYER_N):
        agg4 = _sc_scatter(
            m2.reshape(NQ * N_NODES, QCOLS), src4, dst_flat, zeros)
        w_next = ggc_weight[i + 1] if i + 1 < LAYER_N else None
        if w_next is not None:
            h, m2 = _gru_step(agg4, h, wih_t, whh_t, bih2, bhh2, w_next)
        else:
            h = _gru_step(agg4, h, wih_t, whh_t, bih2, bhh2, None)

    mu, std = _mlp_mix(h, selmask, w1t, mlp_b1, w2t, mlp_b2, w3v, b3row)
    return (mu.reshape(MAX_N, MAX_N, 1), std.reshape(MAX_N, MAX_N, 1))
